# Initial kernel scaffold; baseline (speedup 1.0000x reference)
import jax
import jax.numpy as jnp
from jax.experimental import pallas as pl


def kernel(in_feat, edge_index, e_feat, W0, b0, W1, b1):
    raise NotImplementedError("write your pallas kernel here")


# speedup vs baseline: 4.1681x; 4.1681x over previous
import functools

import jax
import jax.numpy as jnp
from jax import lax
from jax.experimental import pallas as pl
from jax.experimental.pallas import tpu as pltpu
from jax.experimental.pallas import tpu_sc as plsc

N = 10000
E = 320000
D = 128
NPAD = 10240
CHUNK = 128
NCHUNKS = E // CHUNK
NC = 2
NS = 16
NW = NC * NS
CPW = NCHUNKS // NW
CREM = NCHUNKS - CPW * NW
RPW = NPAD // NS


def _worker_chunks(w):
  base = CPW * w + jnp.minimum(w, CREM)
  count = CPW + jnp.where(w < CREM, 1, 0)
  return base, count


def _deg_body(src_hbm, dst_hbm, out_hbm, idx_v, ones_v, zeros_v,
              dsrc_sh, ddst_sh):
  cid = lax.axis_index("c")
  sid = lax.axis_index("s")
  w = cid * NS + sid

  def initz(i, _):
    zeros_v[pl.ds(i * 16, 16)] = jnp.zeros((16,), jnp.float32)
    ones_v[pl.ds(i * 16, 16)] = jnp.ones((16,), jnp.float32)
    return 0
  lax.fori_loop(0, CHUNK // 16, initz, 0)

  def initz2(i, _):
    zeros_v[pl.ds(CHUNK + i * 16, 16)] = jnp.zeros((16,), jnp.float32)
    return 0
  lax.fori_loop(0, (RPW - CHUNK) // 16, initz2, 0)

  pltpu.sync_copy(zeros_v, dsrc_sh.at[pl.ds(sid * RPW, RPW)])
  pltpu.sync_copy(zeros_v, ddst_sh.at[pl.ds(sid * RPW, RPW)])
  plsc.subcore_barrier()

  base, count = _worker_chunks(w)

  def chunk_body(i, _):
    c = base + i
    pltpu.sync_copy(src_hbm.at[c], idx_v.at[0])
    pltpu.sync_copy(ones_v, dsrc_sh.at[idx_v.at[0]], add=True)
    pltpu.sync_copy(dst_hbm.at[c], idx_v.at[0])
    pltpu.sync_copy(ones_v, ddst_sh.at[idx_v.at[0]], add=True)
    return 0
  lax.fori_loop(0, count, chunk_body, 0)

  plsc.subcore_barrier()
  sl = pl.ds(sid * RPW, RPW)
  pltpu.sync_copy(dsrc_sh.at[sl], out_hbm.at[0, cid, sl])
  pltpu.sync_copy(ddst_sh.at[sl], out_hbm.at[1, cid, sl])


def _degrees(src2d, dst2d):
  fn = pl.kernel(
      _deg_body,
      out_type=jax.ShapeDtypeStruct((2, NC, NPAD), jnp.float32),
      mesh=plsc.VectorSubcoreMesh(core_axis_name="c", subcore_axis_name="s"),
      compiler_params=pltpu.CompilerParams(needs_layout_passes=False),
      scratch_types=[
          pltpu.VMEM((1, CHUNK), jnp.int32),
          pltpu.VMEM((CHUNK,), jnp.float32),
          pltpu.VMEM((RPW,), jnp.float32),
          pltpu.VMEM_SHARED((NPAD,), jnp.float32),
          pltpu.VMEM_SHARED((NPAD,), jnp.float32),
      ],
  )
  return fn(src2d, dst2d)


def _scale_body(d_ref, s_ref):
  d = d_ref[...]
  s_ref[...] = lax.rsqrt(jnp.maximum(d[:, 0] + d[:, 1], 1.0))


def _scales(deg_parts):
  d4 = deg_parts.reshape(2, NC, NPAD // 128, 128)
  s = pl.pallas_call(
      _scale_body,
      out_shape=jax.ShapeDtypeStruct((2, NPAD // 128, 128), jnp.float32),
  )(d4)
  return s.reshape(2, NPAD)


def _conv_body(h_hbm, src_hbm, dst_hbm, ef_hbm, sout_hbm, sin_hbm, out_hbm,
               src_v, dst_v, ef_v, w_v, rows_v, sout_v, sin_v, agg_sh, sem):
  cid = lax.axis_index("c")
  sid = lax.axis_index("s")
  w = cid * NS + sid

  pltpu.sync_copy(sout_hbm, sout_v)
  pltpu.sync_copy(sin_hbm, sin_v)

  def zrow(i, _):
    for j in range(D // 16):
      rows_v[i, pl.ds(j * 16, 16)] = jnp.zeros((16,), jnp.float32)
    return 0
  lax.fori_loop(0, CHUNK, zrow, 0)
  for r in range(RPW // CHUNK):
    pltpu.sync_copy(rows_v, agg_sh.at[pl.ds(sid * RPW + r * CHUNK, CHUNK)])
  plsc.subcore_barrier()

  base, count = _worker_chunks(w)

  def chunk_body(i, _):
    c = base + i
    pltpu.sync_copy(src_hbm.at[c], src_v.at[0])
    pltpu.sync_copy(dst_hbm.at[c], dst_v.at[0])
    pltpu.sync_copy(ef_hbm.at[c], ef_v)
    gcp = pltpu.async_copy(h_hbm.at[src_v.at[0]], rows_v, sem)
    for j in range(CHUNK // 16):
      sl = pl.ds(j * 16, 16)
      so = plsc.load_gather(sout_v, [src_v[0, sl]])
      si = plsc.load_gather(sin_v, [dst_v[0, sl]])
      w_v[sl] = ef_v[sl] * so * si
    gcp.wait()

    def erow(e, _):
      we = plsc.load_gather(w_v, [jnp.full((16,), e, jnp.int32)])
      for j in range(D // 16):
        sl = pl.ds(j * 16, 16)
        rows_v[e, sl] = rows_v[e, sl] * we
      return 0
    lax.fori_loop(0, CHUNK, erow, 0)

    pltpu.sync_copy(rows_v, agg_sh.at[dst_v.at[0]], add=True)
    return 0
  lax.fori_loop(0, count, chunk_body, 0)

  plsc.subcore_barrier()
  sl = pl.ds(sid * RPW, RPW)
  pltpu.sync_copy(agg_sh.at[sl], out_hbm.at[cid, sl])


def _conv(h, src2d, dst2d, ef2d, s_out, s_in):
  fn = pl.kernel(
      _conv_body,
      out_type=jax.ShapeDtypeStruct((NC, NPAD, D), jnp.float32),
      mesh=plsc.VectorSubcoreMesh(core_axis_name="c", subcore_axis_name="s"),
      compiler_params=pltpu.CompilerParams(needs_layout_passes=False),
      scratch_types=[
          pltpu.VMEM((1, CHUNK), jnp.int32),
          pltpu.VMEM((1, CHUNK), jnp.int32),
          pltpu.VMEM((CHUNK,), jnp.float32),
          pltpu.VMEM((CHUNK,), jnp.float32),
          pltpu.VMEM((CHUNK, D), jnp.float32),
          pltpu.VMEM((NPAD,), jnp.float32),
          pltpu.VMEM((NPAD,), jnp.float32),
          pltpu.VMEM_SHARED((NPAD, D), jnp.float32),
          pltpu.SemaphoreType.DMA,
      ],
  )
  return fn(h, src2d, dst2d, ef2d, s_out, s_in)


_MM_ROWS = 400


def _mm_body(p_ref, w_ref, b_ref, o_ref):
  p = p_ref[...]
  acc = p[0] + p[1]
  o_ref[...] = jax.lax.dot_general(
      acc, w_ref[...], (((1,), (0,)), ((), ())),
      preferred_element_type=jnp.float32,
      precision=lax.Precision.HIGHEST) + b_ref[...]


def _dense(parts, W, b):
  grid = N // _MM_ROWS
  return pl.pallas_call(
      _mm_body,
      grid=(grid,),
      in_specs=[
          pl.BlockSpec((NC, _MM_ROWS, D), lambda i: (0, i, 0)),
          pl.BlockSpec((D, D), lambda i: (0, 0)),
          pl.BlockSpec((1, D), lambda i: (0, 0)),
      ],
      out_specs=pl.BlockSpec((_MM_ROWS, D), lambda i: (i, 0)),
      out_shape=jax.ShapeDtypeStruct((N, D), jnp.float32),
  )(parts, W, b.reshape(1, D))


def kernel(in_feat, edge_index, e_feat, W0, b0, W1, b1):
  src2d = edge_index[0].reshape(NCHUNKS, CHUNK).astype(jnp.int32)
  dst2d = edge_index[1].reshape(NCHUNKS, CHUNK).astype(jnp.int32)
  ef2d = e_feat.reshape(NCHUNKS, CHUNK)

  deg_parts = _degrees(src2d, dst2d)
  s = _scales(deg_parts)
  s_out, s_in = s[0], s[1]

  parts1 = _conv(in_feat, src2d, dst2d, ef2d, s_out, s_in)
  h1 = _dense(parts1, W0, b0)
  parts2 = _conv(h1, src2d, dst2d, ef2d, s_out, s_in)
  return _dense(parts2, W1, b1)
